# speedup vs baseline: 5.3270x; 5.3270x over previous
import functools

import jax
import jax.numpy as jnp
from jax import lax
from jax.experimental import pallas as pl
from jax.experimental.pallas import tpu as pltpu
from jax.experimental.pallas import tpu_sc as plsc

N = 10000
E = 160000
D = 256
H = 128
NC = 2
NS = 16
K = 80
NCHUNK = 125
NP = 10240
RPTS = NP // NS


def _sc_body(xflat_hbm, srcadj_hbm, dst_hbm, zrows_hbm, zcnt_hbm, ones_hbm,
             sums_hbm, counts_hbm,
             sidx, didx, rows, onesv, ssum, scnt, sem0):
    c = lax.axis_index("c")
    s = lax.axis_index("s")
    wid = c * NS + s

    pltpu.sync_copy(zrows_hbm, ssum.at[pl.ds(s * RPTS, RPTS)])
    pltpu.sync_copy(zcnt_hbm, scnt.at[pl.ds(s * RPTS, RPTS)])

    pltpu.sync_copy(srcadj_hbm.at[wid], sidx)
    pltpu.sync_copy(dst_hbm.at[s], didx)
    pltpu.sync_copy(ones_hbm, onesv)
    plsc.subcore_barrier()

    def chunk(k, carry):
        pltpu.async_copy(xflat_hbm.at[sidx.at[k]], rows, sem0).wait()
        pltpu.sync_copy(rows, ssum.at[didx.at[k]], add=True)
        pltpu.sync_copy(onesv, scnt.at[didx.at[k]], add=True)
        return carry

    lax.fori_loop(0, NCHUNK, chunk, 0)

    plsc.subcore_barrier()

    pltpu.sync_copy(ssum.at[pl.ds(s * RPTS, RPTS)],
                    sums_hbm.at[pl.ds(c * NP + s * RPTS, RPTS)])
    pltpu.sync_copy(scnt.at[pl.ds(s * RPTS, RPTS)],
                    counts_hbm.at[pl.ds(c * NP + s * RPTS, RPTS)])


def _sc_aggregate(xflat, srcadj, dst_rs, zrows, zcnt, ones):
    mesh = plsc.VectorSubcoreMesh(core_axis_name="c", subcore_axis_name="s")
    return pl.kernel(
        _sc_body,
        out_type=[
            jax.ShapeDtypeStruct((NC * NP, H), jnp.float32),
            jax.ShapeDtypeStruct((NC * NP,), jnp.float32),
        ],
        mesh=mesh,
        scratch_types=[
            pltpu.VMEM((NCHUNK, K), jnp.int32),
            pltpu.VMEM((NCHUNK, K), jnp.int32),
            pltpu.VMEM((K, H), jnp.float32),
            pltpu.VMEM((K,), jnp.float32),
            pltpu.VMEM_SHARED((NP, H), jnp.float32),
            pltpu.VMEM_SHARED((NP,), jnp.float32),
            pltpu.SemaphoreType.DMA,
        ],
    )(xflat, srcadj, dst_rs, zrows, zcnt, ones)


def _tc_body(x_ref, sums_ref, cnt_ref, w1t_ref, w2at_ref, w2bt_ref,
             b_ref, gamma_ref, beta_ref, out_ref):
    x = x_ref[...]
    s0 = sums_ref[pl.ds(0, N), :]
    s1 = sums_ref[pl.ds(NP, N), :]
    rec = 1.0 / jnp.maximum(cnt_ref[...], 1.0)
    m = jnp.dot(x, w1t_ref[...], preferred_element_type=jnp.float32)
    agg = (jnp.dot(s0, w2at_ref[...], preferred_element_type=jnp.float32)
           + jnp.dot(s1, w2bt_ref[...], preferred_element_type=jnp.float32))
    m = m + agg * rec + b_ref[...]
    mean = jnp.mean(m, axis=0, keepdims=True)
    d = m - mean
    var = jnp.mean(d * d, axis=0, keepdims=True)
    y = d * lax.rsqrt(var + 1e-5) * gamma_ref[...] + beta_ref[...]
    out_ref[...] = 0.5 * y * (1.0 + lax.erf(y * 0.7071067811865475))


def _tc_fused(x, sums_all, cnt, w1t, w2at, w2bt, b2, gamma2, beta2):
    return pl.pallas_call(
        _tc_body,
        out_shape=jax.ShapeDtypeStruct((N, D), jnp.float32),
    )(x, sums_all, cnt, w1t, w2at, w2bt, b2, gamma2, beta2)


@jax.jit
def kernel(x, edge_index, W, b, gamma, beta):
    src = edge_index[0]
    dst = edge_index[1]

    xflat = jnp.concatenate([x[:, :H], x[:, H:]], axis=0)
    src_rs = src.reshape(NS, NCHUNK, K)
    srcadj = jnp.concatenate([src_rs, src_rs + N], axis=0)
    dst_rs = dst.reshape(NS, NCHUNK, K)
    zrows = jnp.zeros((RPTS, H), jnp.float32)
    zcnt = jnp.zeros((RPTS,), jnp.float32)
    ones = jnp.ones((K,), jnp.float32)

    sums_all, counts_all = _sc_aggregate(xflat, srcadj, dst_rs,
                                         zrows, zcnt, ones)

    cnt = counts_all[:N][:, None]
    w1t = W[:, :D].T
    w2at = W[:, D:D + H].T
    w2bt = W[:, D + H:].T
    return _tc_fused(x, sums_all, cnt, w1t, w2at, w2bt,
                     b[None, :], gamma[None, :], beta[None, :])


# speedup vs baseline: 6.5631x; 1.2320x over previous
import functools

import jax
import jax.numpy as jnp
from jax import lax
from jax.experimental import pallas as pl
from jax.experimental.pallas import tpu as pltpu
from jax.experimental.pallas import tpu_sc as plsc

N = 10000
E = 160000
D = 256
H = 128
NC = 2
NS = 16
K = 80
NCHUNK = 125
NP = 10240
RPTS = NP // NS


def _sc_body(xflat_hbm, idxcat_hbm, zrows_hbm, zcnt_hbm, ones_hbm,
             sums_hbm, counts_hbm,
             ibuf, rows, onesv, ssum, scnt, sem0, sem1):
    c = lax.axis_index("c")
    s = lax.axis_index("s")
    wid = c * NS + s

    pltpu.sync_copy(zrows_hbm, ssum.at[pl.ds(s * RPTS, RPTS)])
    pltpu.sync_copy(zcnt_hbm, scnt.at[pl.ds(s * RPTS, RPTS)])
    pltpu.sync_copy(ones_hbm, onesv)
    plsc.subcore_barrier()

    def load_idx(k, buf):
        pltpu.sync_copy(idxcat_hbm.at[wid, k], ibuf.at[buf])

    def gather(buf, sem):
        return pltpu.async_copy(xflat_hbm.at[ibuf.at[buf, 0]],
                                rows.at[buf], sem)

    def wait_gather(buf, sem):
        pltpu.make_async_copy(xflat_hbm.at[ibuf.at[buf, 0]],
                              rows.at[buf], sem).wait()

    def scatter(buf):
        pltpu.sync_copy(rows.at[buf], ssum.at[ibuf.at[buf, 1]], add=True)
        pltpu.sync_copy(onesv, scnt.at[ibuf.at[buf, 1]], add=True)

    load_idx(0, 0)
    gather(0, sem0)

    def pair(i, carry):
        k0 = 2 * i
        load_idx(k0 + 1, 1)
        cp1 = gather(1, sem1)
        wait_gather(0, sem0)
        scatter(0)
        load_idx(k0 + 2, 0)
        cp1.wait()
        gather(0, sem0)
        scatter(1)
        return carry

    lax.fori_loop(0, (NCHUNK - 1) // 2, pair, 0)
    wait_gather(0, sem0)
    scatter(0)

    plsc.subcore_barrier()

    pltpu.sync_copy(ssum.at[pl.ds(s * RPTS, RPTS)],
                    sums_hbm.at[pl.ds(c * NP + s * RPTS, RPTS)])
    pltpu.sync_copy(scnt.at[pl.ds(s * RPTS, RPTS)],
                    counts_hbm.at[pl.ds(c * NP + s * RPTS, RPTS)])


def _sc_aggregate(xflat, idxcat, zrows, zcnt, ones):
    mesh = plsc.VectorSubcoreMesh(core_axis_name="c", subcore_axis_name="s")
    return pl.kernel(
        _sc_body,
        out_type=[
            jax.ShapeDtypeStruct((NC * NP, H), jnp.float32),
            jax.ShapeDtypeStruct((NC * NP,), jnp.float32),
        ],
        mesh=mesh,
        scratch_types=[
            pltpu.VMEM((2, 2, K), jnp.int32),
            pltpu.VMEM((2, K, H), jnp.float32),
            pltpu.VMEM((K,), jnp.float32),
            pltpu.VMEM_SHARED((NP, H), jnp.float32),
            pltpu.VMEM_SHARED((NP,), jnp.float32),
            pltpu.SemaphoreType.DMA,
            pltpu.SemaphoreType.DMA,
        ],
    )(xflat, idxcat, zrows, zcnt, ones)


def _tc_body(x_ref, sums_ref, cnt_ref, w1t_ref, w2at_ref, w2bt_ref,
             b_ref, gamma_ref, beta_ref, out_ref):
    x = x_ref[...]
    s0 = sums_ref[pl.ds(0, N), :]
    s1 = sums_ref[pl.ds(NP, N), :]
    rec = 1.0 / jnp.maximum(cnt_ref[...], 1.0)
    m = jnp.dot(x, w1t_ref[...], preferred_element_type=jnp.float32)
    agg = (jnp.dot(s0, w2at_ref[...], preferred_element_type=jnp.float32)
           + jnp.dot(s1, w2bt_ref[...], preferred_element_type=jnp.float32))
    m = m + agg * rec + b_ref[...]
    mean = jnp.mean(m, axis=0, keepdims=True)
    d = m - mean
    var = jnp.mean(d * d, axis=0, keepdims=True)
    y = d * lax.rsqrt(var + 1e-5) * gamma_ref[...] + beta_ref[...]
    out_ref[...] = 0.5 * y * (1.0 + lax.erf(y * 0.7071067811865475))


def _tc_fused(x, sums_all, cnt, w1t, w2at, w2bt, b2, gamma2, beta2):
    return pl.pallas_call(
        _tc_body,
        out_shape=jax.ShapeDtypeStruct((N, D), jnp.float32),
    )(x, sums_all, cnt, w1t, w2at, w2bt, b2, gamma2, beta2)


@jax.jit
def kernel(x, edge_index, W, b, gamma, beta):
    src = edge_index[0]
    dst = edge_index[1]

    xflat = jnp.concatenate([x[:, :H], x[:, H:]], axis=0)
    src_rs = src.reshape(NS, NCHUNK, K)
    srcadj = jnp.concatenate([src_rs, src_rs + N], axis=0)
    dst_rs = dst.reshape(NS, NCHUNK, K)
    dst2 = jnp.concatenate([dst_rs, dst_rs], axis=0)
    idxcat = jnp.stack([srcadj, dst2], axis=2)
    zrows = jnp.zeros((RPTS, H), jnp.float32)
    zcnt = jnp.zeros((RPTS,), jnp.float32)
    ones = jnp.ones((K,), jnp.float32)

    sums_all, counts_all = _sc_aggregate(xflat, idxcat, zrows, zcnt, ones)

    cnt = counts_all[:N][:, None]
    w1t = W[:, :D].T
    w2at = W[:, D:D + H].T
    w2bt = W[:, D + H:].T
    return _tc_fused(x, sums_all, cnt, w1t, w2at, w2bt,
                     b[None, :], gamma[None, :], beta[None, :])


# speedup vs baseline: 6.6956x; 1.0202x over previous
import functools

import jax
import jax.numpy as jnp
from jax import lax
from jax.experimental import pallas as pl
from jax.experimental.pallas import tpu as pltpu
from jax.experimental.pallas import tpu_sc as plsc

N = 10000
E = 160000
D = 256
H = 128
NC = 2
NS = 16
K = 80
NCHUNK = 125
NP = 10240
RPTS = NP // NS


def _sc_body(xflat_hbm, idxcat_hbm, zrows_hbm, zcnt_hbm, ones_hbm,
             sums_hbm, counts_hbm,
             ibuf, rows, onesv, ssum, scnt,
             semg0, semg1, sems0, sems1, semc0, semc1):
    c = lax.axis_index("c")
    s = lax.axis_index("s")
    wid = c * NS + s
    semg = (semg0, semg1)
    sems = (sems0, sems1)
    semc = (semc0, semc1)

    pltpu.sync_copy(zrows_hbm, ssum.at[pl.ds(s * RPTS, RPTS)])
    pltpu.sync_copy(zcnt_hbm, scnt.at[pl.ds(s * RPTS, RPTS)])
    pltpu.sync_copy(ones_hbm, onesv)
    plsc.subcore_barrier()

    def load_idx(k, buf):
        pltpu.sync_copy(idxcat_hbm.at[wid, k], ibuf.at[buf])

    def gather(buf):
        pltpu.async_copy(xflat_hbm.at[ibuf.at[buf, 0]],
                         rows.at[buf], semg[buf])

    def wait_gather(buf):
        pltpu.make_async_copy(xflat_hbm.at[ibuf.at[buf, 0]],
                              rows.at[buf], semg[buf]).wait()

    def scatter(buf):
        pltpu.async_copy(rows.at[buf], ssum.at[ibuf.at[buf, 1]],
                         sems[buf], add=True)
        pltpu.async_copy(onesv, scnt.at[ibuf.at[buf, 1]],
                         semc[buf], add=True)

    def wait_scatter(buf):
        pltpu.make_async_copy(rows.at[buf], ssum.at[ibuf.at[buf, 1]],
                              sems[buf]).wait()
        pltpu.make_async_copy(onesv, scnt.at[ibuf.at[buf, 1]],
                              semc[buf]).wait()

    def first_half(k0):
        load_idx(k0 + 1, 1)
        gather(1)
        wait_gather(0)
        scatter(0)

    def second_half(k0):
        wait_scatter(0)
        load_idx(k0 + 2, 0)
        gather(0)
        wait_gather(1)
        scatter(1)

    load_idx(0, 0)
    gather(0)
    first_half(0)
    second_half(0)

    def pair(i, carry):
        k0 = 2 * i
        wait_scatter(1)
        first_half(k0)
        second_half(k0)
        return carry

    lax.fori_loop(1, (NCHUNK - 1) // 2, pair, 0)
    wait_scatter(1)
    wait_gather(0)
    scatter(0)
    wait_scatter(0)

    plsc.subcore_barrier()

    pltpu.sync_copy(ssum.at[pl.ds(s * RPTS, RPTS)],
                    sums_hbm.at[pl.ds(c * NP + s * RPTS, RPTS)])
    pltpu.sync_copy(scnt.at[pl.ds(s * RPTS, RPTS)],
                    counts_hbm.at[pl.ds(c * NP + s * RPTS, RPTS)])


def _sc_aggregate(xflat, idxcat, zrows, zcnt, ones):
    mesh = plsc.VectorSubcoreMesh(core_axis_name="c", subcore_axis_name="s")
    return pl.kernel(
        _sc_body,
        out_type=[
            jax.ShapeDtypeStruct((NC * NP, H), jnp.float32),
            jax.ShapeDtypeStruct((NC * NP,), jnp.float32),
        ],
        mesh=mesh,
        scratch_types=[
            pltpu.VMEM((2, 2, K), jnp.int32),
            pltpu.VMEM((2, K, H), jnp.float32),
            pltpu.VMEM((K,), jnp.float32),
            pltpu.VMEM_SHARED((NP, H), jnp.float32),
            pltpu.VMEM_SHARED((NP,), jnp.float32),
            pltpu.SemaphoreType.DMA,
            pltpu.SemaphoreType.DMA,
            pltpu.SemaphoreType.DMA,
            pltpu.SemaphoreType.DMA,
            pltpu.SemaphoreType.DMA,
            pltpu.SemaphoreType.DMA,
        ],
    )(xflat, idxcat, zrows, zcnt, ones)


def _tc_body(x_ref, sums_ref, cnt_ref, w1t_ref, w2at_ref, w2bt_ref,
             b_ref, gamma_ref, beta_ref, out_ref):
    x = x_ref[...]
    s0 = sums_ref[pl.ds(0, N), :]
    s1 = sums_ref[pl.ds(NP, N), :]
    rec = 1.0 / jnp.maximum(cnt_ref[...], 1.0)
    m = jnp.dot(x, w1t_ref[...], preferred_element_type=jnp.float32)
    agg = (jnp.dot(s0, w2at_ref[...], preferred_element_type=jnp.float32)
           + jnp.dot(s1, w2bt_ref[...], preferred_element_type=jnp.float32))
    m = m + agg * rec + b_ref[...]
    mean = jnp.mean(m, axis=0, keepdims=True)
    d = m - mean
    var = jnp.mean(d * d, axis=0, keepdims=True)
    y = d * lax.rsqrt(var + 1e-5) * gamma_ref[...] + beta_ref[...]
    out_ref[...] = 0.5 * y * (1.0 + lax.erf(y * 0.7071067811865475))


def _tc_fused(x, sums_all, cnt, w1t, w2at, w2bt, b2, gamma2, beta2):
    return pl.pallas_call(
        _tc_body,
        out_shape=jax.ShapeDtypeStruct((N, D), jnp.float32),
    )(x, sums_all, cnt, w1t, w2at, w2bt, b2, gamma2, beta2)


@jax.jit
def kernel(x, edge_index, W, b, gamma, beta):
    src = edge_index[0]
    dst = edge_index[1]

    xflat = jnp.concatenate([x[:, :H], x[:, H:]], axis=0)
    src_rs = src.reshape(NS, NCHUNK, K)
    srcadj = jnp.concatenate([src_rs, src_rs + N], axis=0)
    dst_rs = dst.reshape(NS, NCHUNK, K)
    dst2 = jnp.concatenate([dst_rs, dst_rs], axis=0)
    idxcat = jnp.stack([srcadj, dst2], axis=2)
    zrows = jnp.zeros((RPTS, H), jnp.float32)
    zcnt = jnp.zeros((RPTS,), jnp.float32)
    ones = jnp.ones((K,), jnp.float32)

    sums_all, counts_all = _sc_aggregate(xflat, idxcat, zrows, zcnt, ones)

    cnt = counts_all[:N][:, None]
    w1t = W[:, :D].T
    w2at = W[:, D:D + H].T
    w2bt = W[:, D + H:].T
    return _tc_fused(x, sums_all, cnt, w1t, w2at, w2bt,
                     b[None, :], gamma[None, :], beta[None, :])
